# Initial kernel scaffold; baseline (speedup 1.0000x reference)
import jax
import jax.numpy as jnp
from jax.experimental import pallas as pl


def kernel(x, W1, b1, W2, b2, W3, b3):
    raise NotImplementedError("write your pallas kernel here")


# speedup vs baseline: 3.6503x; 3.6503x over previous
import jax
import jax.numpy as jnp
from jax.experimental import pallas as pl
from jax.experimental.pallas import tpu as pltpu


def _mlp_body(nidx_ref, a_ref, ash_ref, w1_ref, b1_ref, w2_ref, b2_ref,
              w3_ref, b3_ref, o_ref):
    f = pl.program_id(0)
    bdim = a_ref.shape[0]
    thr = f * bdim
    xf = jnp.where(nidx_ref[...] < thr, a_ref[...], ash_ref[...])
    h = jnp.dot(xf, w1_ref[0], preferred_element_type=jnp.float32) + b1_ref[0]
    h = jnp.maximum(h, 0.0)
    h = jnp.dot(h, w2_ref[0], preferred_element_type=jnp.float32) + b2_ref[0]
    h = jnp.maximum(h, 0.0)
    h = jnp.dot(h, w3_ref[0], preferred_element_type=jnp.float32) + b3_ref[0]
    h = jnp.maximum(h, 0.0)
    o_ref[...] = h[None]


def kernel(x, W1, b1, W2, b2, W3, b3):
    F, B = x.shape
    K = F - 1
    H = b1.shape[-1]
    xf = x.reshape(-1)
    A = xf[:B * K].reshape(B, K)
    Ash = xf[B:B + B * K].reshape(B, K)
    nidx = (jnp.arange(B, dtype=jnp.int32)[:, None] * K
            + jnp.arange(K, dtype=jnp.int32)[None, :])
    b1r = b1.reshape(F, 1, H)
    b2r = b2.reshape(F, 1, H)
    b3r = b3.reshape(F, 1, H)

    out = pl.pallas_call(
        _mlp_body,
        grid=(F,),
        in_specs=[
            pl.BlockSpec((B, K), lambda f: (0, 0)),
            pl.BlockSpec((B, K), lambda f: (0, 0)),
            pl.BlockSpec((B, K), lambda f: (0, 0)),
            pl.BlockSpec((1, K, H), lambda f: (f, 0, 0)),
            pl.BlockSpec((1, 1, H), lambda f: (f, 0, 0)),
            pl.BlockSpec((1, H, H), lambda f: (f, 0, 0)),
            pl.BlockSpec((1, 1, H), lambda f: (f, 0, 0)),
            pl.BlockSpec((1, H, H), lambda f: (f, 0, 0)),
            pl.BlockSpec((1, 1, H), lambda f: (f, 0, 0)),
        ],
        out_specs=pl.BlockSpec((1, B, H), lambda f: (f, 0, 0)),
        out_shape=jax.ShapeDtypeStruct((F, B, H), jnp.float32),
        compiler_params=pltpu.CompilerParams(
            dimension_semantics=("parallel",),
        ),
        name="feature_loo_mlp",
    )(nidx, A, Ash, W1, b1r, W2, b2r, W3, b3r)
    return out.reshape(-1)


# speedup vs baseline: 4.0256x; 1.1028x over previous
import jax
import jax.numpy as jnp
from jax import lax
from jax.experimental import pallas as pl
from jax.experimental.pallas import tpu as pltpu

_G = 8


def _body(a_ref, ash_ref, w1_ref, b1_ref, w2_ref, b2_ref, w3_ref, b3_ref,
          bfrow_ref, base_ref, thr_ref, o_ref, h1s_ref, xfix_ref):
    s = pl.program_id(0)
    B, K = a_ref.shape
    NW = w1_ref.shape[1]
    H = NW // _G

    p = jnp.dot(a_ref[...], w1_ref[...], preferred_element_type=jnp.float32)
    q = jnp.dot(ash_ref[...], w1_ref[...], preferred_element_type=jnp.float32)
    bf = bfrow_ref[0]
    rows = lax.broadcasted_iota(jnp.int32, (B, NW), 0)
    h1s_ref[...] = jnp.where(rows < bf, p, q)

    for g in range(_G):
        base = pl.multiple_of(base_ref[s, g], 8)
        thr = thr_ref[s, g]
        a8 = a_ref[pl.ds(base, 8), :]
        ash8 = ash_ref[pl.ds(base, 8), :]
        n8 = ((lax.broadcasted_iota(jnp.int32, (8, K), 0) + base) * K
              + lax.broadcasted_iota(jnp.int32, (8, K), 1))
        xfix_ref[8 * g:8 * (g + 1), :] = jnp.where(n8 < thr, a8, ash8)
    fix = jnp.dot(xfix_ref[...], w1_ref[...],
                  preferred_element_type=jnp.float32)
    lane = lax.broadcasted_iota(jnp.int32, (8, NW), 1)
    for g in range(_G):
        base = pl.multiple_of(base_ref[s, g], 8)
        win = h1s_ref[pl.ds(base, 8), :]
        m = (lane >= g * H) & (lane < (g + 1) * H)
        h1s_ref[pl.ds(base, 8), :] = jnp.where(m, fix[8 * g:8 * (g + 1), :], win)

    h = jnp.maximum(h1s_ref[...] + b1_ref[...], 0.0)
    h = jnp.dot(h, w2_ref[0], preferred_element_type=jnp.float32) + b2_ref[...]
    h = jnp.maximum(h, 0.0)
    h = jnp.dot(h, w3_ref[0], preferred_element_type=jnp.float32) + b3_ref[...]
    h = jnp.maximum(h, 0.0)
    o_ref[...] = h[None]


def kernel(x, W1, b1, W2, b2, W3, b3):
    F, B = x.shape
    K = F - 1
    H = b1.shape[-1]
    G = _G
    S = -(-F // G)
    Fp = S * G
    pad = Fp - F
    NW = G * H

    xf = x.reshape(-1)
    A = xf[:B * K].reshape(B, K)
    Ash = xf[B:B + B * K].reshape(B, K)

    W1p = jnp.pad(W1, ((0, pad), (0, 0), (0, 0)))
    b1p = jnp.pad(b1, ((0, pad), (0, 0)))
    W2p = jnp.pad(W2, ((0, pad), (0, 0), (0, 0)))
    b2p = jnp.pad(b2, ((0, pad), (0, 0)))
    W3p = jnp.pad(W3, ((0, pad), (0, 0), (0, 0)))
    b3p = jnp.pad(b3, ((0, pad), (0, 0)))

    W1c = W1p.transpose(1, 0, 2).reshape(K, Fp * H)
    b1c = b1p.reshape(1, Fp * H)
    b2c = b2p.reshape(1, Fp * H)
    b3c = b3p.reshape(1, Fp * H)
    eye = jnp.eye(G, dtype=W2.dtype)
    W2b = jnp.einsum('sgij,gh->sgihj', W2p.reshape(S, G, H, H),
                     eye).reshape(S, NW, NW)
    W3b = jnp.einsum('sgij,gh->sgihj', W3p.reshape(S, G, H, H),
                     eye).reshape(S, NW, NW)

    f_all = jnp.arange(Fp, dtype=jnp.int32)
    t_all = f_all * B
    bf_all = t_all // K
    bfrow = jnp.repeat(bf_all.reshape(S, G), H, axis=1).reshape(S, 1, NW)
    basearr = jnp.minimum((bf_all >> 3) << 3, B - 8).reshape(S, G)
    thrarr = t_all.reshape(S, G)

    out = pl.pallas_call(
        _body,
        grid=(S,),
        in_specs=[
            pl.BlockSpec((B, K), lambda s: (0, 0)),
            pl.BlockSpec((B, K), lambda s: (0, 0)),
            pl.BlockSpec((K, NW), lambda s: (0, s)),
            pl.BlockSpec((1, NW), lambda s: (0, s)),
            pl.BlockSpec((1, NW, NW), lambda s: (s, 0, 0)),
            pl.BlockSpec((1, NW), lambda s: (0, s)),
            pl.BlockSpec((1, NW, NW), lambda s: (s, 0, 0)),
            pl.BlockSpec((1, NW), lambda s: (0, s)),
            pl.BlockSpec((1, 1, NW), lambda s: (s, 0, 0)),
            pl.BlockSpec(memory_space=pltpu.SMEM),
            pl.BlockSpec(memory_space=pltpu.SMEM),
        ],
        out_specs=pl.BlockSpec((1, B, NW), lambda s: (s, 0, 0)),
        out_shape=jax.ShapeDtypeStruct((S, B, NW), jnp.float32),
        scratch_shapes=[
            pltpu.VMEM((B, NW), jnp.float32),
            pltpu.VMEM((8 * G, K), jnp.float32),
        ],
        compiler_params=pltpu.CompilerParams(
            dimension_semantics=("arbitrary",),
        ),
        name="feature_loo_mlp_g8",
    )(A, Ash, W1c, b1c, W2b, b2c, W3b, b3c, bfrow, basearr, thrarr)

    out = out.reshape(S, B, G, H).transpose(0, 2, 1, 3).reshape(Fp, B, H)
    if pad:
        out = out[:F]
    return out.reshape(-1)


# speedup vs baseline: 8.8546x; 2.1995x over previous
import jax
import jax.numpy as jnp
from jax import lax
from jax.experimental import pallas as pl
from jax.experimental.pallas import tpu as pltpu


def _pick_group(F, H):
    for d in range(1, F + 1):
        if F % d == 0 and d * H >= 256:
            return d
    return F


def _body(a_ref, ash_ref, w1_ref, b1_ref, w2_ref, b2_ref, w3_ref, b3_ref,
          bfrow_ref, base_ref, thr_ref, o_ref,
          h1s_ref, xfix_ref, w1c_ref, w2d_ref, w3d_ref):
    s = pl.program_id(0)
    B, K = a_ref.shape
    G = w1_ref.shape[0]
    H = w1_ref.shape[2]
    NW = G * H
    J = 128 // H
    B4 = B // J

    w1c_ref[...] = jnp.concatenate([w1_ref[g] for g in range(G)], axis=1)

    @pl.when(s == 0)
    def _zero_diag():
        w2d_ref[...] = jnp.zeros_like(w2d_ref)
        w3d_ref[...] = jnp.zeros_like(w3d_ref)

    for g in range(G):
        w2d_ref[g * H:(g + 1) * H, g * H:(g + 1) * H] = w2_ref[g]
        w3d_ref[g * H:(g + 1) * H, g * H:(g + 1) * H] = w3_ref[g]

    w1c = w1c_ref[...]
    p = jnp.dot(a_ref[...], w1c, preferred_element_type=jnp.float32)
    q = jnp.dot(ash_ref[...], w1c, preferred_element_type=jnp.float32)
    bf = bfrow_ref[0]
    rows = lax.broadcasted_iota(jnp.int32, (B, NW), 0)
    h1s_ref[...] = jnp.where(rows < bf, p, q)

    for g in range(G):
        base = pl.multiple_of(base_ref[s, g], 8)
        thr = thr_ref[s, g]
        a8 = a_ref[pl.ds(base, 8), :]
        ash8 = ash_ref[pl.ds(base, 8), :]
        n8 = ((lax.broadcasted_iota(jnp.int32, (8, K), 0) + base) * K
              + lax.broadcasted_iota(jnp.int32, (8, K), 1))
        xfix_ref[8 * g:8 * (g + 1), :] = jnp.where(n8 < thr, a8, ash8)
    fix = jnp.dot(xfix_ref[...], w1c,
                  preferred_element_type=jnp.float32)
    lane = lax.broadcasted_iota(jnp.int32, (8, NW), 1)
    for g in range(G):
        base = pl.multiple_of(base_ref[s, g], 8)
        win = h1s_ref[pl.ds(base, 8), :]
        m = (lane >= g * H) & (lane < (g + 1) * H)
        h1s_ref[pl.ds(base, 8), :] = jnp.where(m, fix[8 * g:8 * (g + 1), :], win)

    h = jnp.maximum(h1s_ref[...] + b1_ref[0], 0.0)
    h = jnp.dot(h, w2d_ref[...], preferred_element_type=jnp.float32) + b2_ref[0]
    h = jnp.maximum(h, 0.0)
    h = jnp.dot(h, w3d_ref[...], preferred_element_type=jnp.float32) + b3_ref[0]
    h = jnp.maximum(h, 0.0)

    for g in range(G):
        hcol = h[:, g * H:(g + 1) * H]
        hr = hcol.reshape(B4, J, H)
        t = jnp.concatenate([hr[:, j, :] for j in range(J)], axis=1)
        o_ref[g * B4:(g + 1) * B4, :] = t


def kernel(x, W1, b1, W2, b2, W3, b3):
    F, B = x.shape
    K = F - 1
    H = b1.shape[-1]
    G = _pick_group(F, H)
    S = F // G
    NW = G * H
    J = 128 // H
    B4 = B // J

    xf = x.reshape(-1)
    A = xf[:B * K].reshape(B, K)
    Ash = xf[B:B + B * K].reshape(B, K)

    b1c = b1.reshape(S, 1, NW)
    b2c = b2.reshape(S, 1, NW)
    b3c = b3.reshape(S, 1, NW)

    f_all = jnp.arange(F, dtype=jnp.int32)
    t_all = f_all * B
    bf_all = t_all // K
    bfrow = jnp.repeat(bf_all.reshape(S, G), H, axis=1).reshape(S, 1, NW)
    basearr = jnp.minimum((bf_all >> 3) << 3, B - 8).reshape(S, G)
    thrarr = t_all.reshape(S, G)

    out = pl.pallas_call(
        _body,
        grid=(S,),
        in_specs=[
            pl.BlockSpec((B, K), lambda s: (0, 0)),
            pl.BlockSpec((B, K), lambda s: (0, 0)),
            pl.BlockSpec((G, K, H), lambda s: (s, 0, 0)),
            pl.BlockSpec((1, 1, NW), lambda s: (s, 0, 0)),
            pl.BlockSpec((G, H, H), lambda s: (s, 0, 0)),
            pl.BlockSpec((1, 1, NW), lambda s: (s, 0, 0)),
            pl.BlockSpec((G, H, H), lambda s: (s, 0, 0)),
            pl.BlockSpec((1, 1, NW), lambda s: (s, 0, 0)),
            pl.BlockSpec((1, 1, NW), lambda s: (s, 0, 0)),
            pl.BlockSpec(memory_space=pltpu.SMEM),
            pl.BlockSpec(memory_space=pltpu.SMEM),
        ],
        out_specs=pl.BlockSpec((G * B4, 128), lambda s: (s, 0)),
        out_shape=jax.ShapeDtypeStruct((F * B4, 128), jnp.float32),
        scratch_shapes=[
            pltpu.VMEM((B, NW), jnp.float32),
            pltpu.VMEM((8 * G, K), jnp.float32),
            pltpu.VMEM((K, NW), jnp.float32),
            pltpu.VMEM((NW, NW), jnp.float32),
            pltpu.VMEM((NW, NW), jnp.float32),
        ],
        compiler_params=pltpu.CompilerParams(
            dimension_semantics=("arbitrary",),
        ),
        name="feature_loo_mlp_g10",
    )(A, Ash, W1, b1c, W2, b2c, W3, b3c, bfrow, basearr, thrarr)

    return out.reshape(-1)


# speedup vs baseline: 8.9173x; 1.0071x over previous
import jax
import jax.numpy as jnp
from jax import lax
from jax.experimental import pallas as pl
from jax.experimental.pallas import tpu as pltpu


def _pick_group(F, H):
    for d in range(1, F + 1):
        if F % d == 0 and d * H >= 256:
            return d
    return F


def _body(a_ref, ash_ref, w1_ref, b1_ref, w2_ref, b2_ref, w3_ref, b3_ref,
          bfrow_ref, rp_ref, base_ref, thr_ref, o_ref,
          h1s_ref, xfix_ref, w1c_ref, w2d_ref, w3d_ref):
    s = pl.program_id(0)
    B, K = a_ref.shape
    G = w1_ref.shape[0]
    H = w1_ref.shape[2]
    NW = G * H
    J = min(128 // H, B // 8)
    B4 = B // J

    w1c_ref[...] = jnp.concatenate([w1_ref[g] for g in range(G)], axis=1)

    @pl.when(s == 0)
    def _zero_diag():
        w2d_ref[...] = jnp.zeros_like(w2d_ref)
        w3d_ref[...] = jnp.zeros_like(w3d_ref)

    for g in range(G):
        w2d_ref[g * H:(g + 1) * H, g * H:(g + 1) * H] = w2_ref[g]
        w3d_ref[g * H:(g + 1) * H, g * H:(g + 1) * H] = w3_ref[g]

    w1c = w1c_ref[...]
    p = jnp.dot(a_ref[...], w1c, preferred_element_type=jnp.float32)
    q = jnp.dot(ash_ref[...], w1c, preferred_element_type=jnp.float32)
    bf = bfrow_ref[0]
    rp = rp_ref[...]
    h1s_ref[...] = jnp.where(rp < bf, p, q)

    for g in range(G):
        base = pl.multiple_of(base_ref[s, g], 8 * J)
        thr = thr_ref[s, g]
        r0 = pl.multiple_of(base // J, 8)
        for j in range(J):
            apj = a_ref[pl.ds(r0 + B4 * j, 8), :]
            ashpj = ash_ref[pl.ds(r0 + B4 * j, 8), :]
            borig = base + J * lax.broadcasted_iota(jnp.int32, (8, K), 0) + j
            n8 = borig * K + lax.broadcasted_iota(jnp.int32, (8, K), 1)
            xfix_ref[8 * (J * g + j):8 * (J * g + j + 1), :] = (
                jnp.where(n8 < thr, apj, ashpj))
    fix = jnp.dot(xfix_ref[...], w1c,
                  preferred_element_type=jnp.float32)
    lane = lax.broadcasted_iota(jnp.int32, (8, NW), 1)
    for g in range(G):
        base = pl.multiple_of(base_ref[s, g], 8 * J)
        r0 = pl.multiple_of(base // J, 8)
        m = (lane >= g * H) & (lane < (g + 1) * H)
        for j in range(J):
            win = h1s_ref[pl.ds(r0 + B4 * j, 8), :]
            h1s_ref[pl.ds(r0 + B4 * j, 8), :] = jnp.where(
                m, fix[8 * (J * g + j):8 * (J * g + j + 1), :], win)

    h = jnp.maximum(h1s_ref[...] + b1_ref[0], 0.0)
    h = jnp.dot(h, w2d_ref[...], preferred_element_type=jnp.float32) + b2_ref[0]
    h = jnp.maximum(h, 0.0)
    h = jnp.dot(h, w3d_ref[...], preferred_element_type=jnp.float32) + b3_ref[0]
    h = jnp.maximum(h, 0.0)

    for g in range(G):
        t = jnp.concatenate(
            [h[B4 * j:B4 * (j + 1), g * H:(g + 1) * H] for j in range(J)],
            axis=1)
        o_ref[g * B4:(g + 1) * B4, :] = t


def kernel(x, W1, b1, W2, b2, W3, b3):
    F, B = x.shape
    K = F - 1
    H = b1.shape[-1]
    G = _pick_group(F, H)
    S = F // G
    NW = G * H
    J = min(128 // H, B // 8)
    B4 = B // J

    xf = x.reshape(-1)
    A = xf[:B * K].reshape(B, K)
    Ash = xf[B:B + B * K].reshape(B, K)
    Ap = A.reshape(B4, J, K).transpose(1, 0, 2).reshape(B, K)
    Ashp = Ash.reshape(B4, J, K).transpose(1, 0, 2).reshape(B, K)
    permvec = (J * (jnp.arange(B, dtype=jnp.int32) % B4)
               + jnp.arange(B, dtype=jnp.int32) // B4)
    rp = jnp.broadcast_to(permvec[:, None], (B, NW))

    b1c = b1.reshape(S, 1, NW)
    b2c = b2.reshape(S, 1, NW)
    b3c = b3.reshape(S, 1, NW)

    f_all = jnp.arange(F, dtype=jnp.int32)
    t_all = f_all * B
    bf_all = t_all // K
    bfrow = jnp.repeat(bf_all.reshape(S, G), H, axis=1).reshape(S, 1, NW)
    W8 = 8 * J
    basearr = jnp.minimum((bf_all // W8) * W8, B - W8).reshape(S, G)
    thrarr = t_all.reshape(S, G)

    out = pl.pallas_call(
        _body,
        grid=(S,),
        in_specs=[
            pl.BlockSpec((B, K), lambda s: (0, 0)),
            pl.BlockSpec((B, K), lambda s: (0, 0)),
            pl.BlockSpec((G, K, H), lambda s: (s, 0, 0)),
            pl.BlockSpec((1, 1, NW), lambda s: (s, 0, 0)),
            pl.BlockSpec((G, H, H), lambda s: (s, 0, 0)),
            pl.BlockSpec((1, 1, NW), lambda s: (s, 0, 0)),
            pl.BlockSpec((G, H, H), lambda s: (s, 0, 0)),
            pl.BlockSpec((1, 1, NW), lambda s: (s, 0, 0)),
            pl.BlockSpec((1, 1, NW), lambda s: (s, 0, 0)),
            pl.BlockSpec((B, NW), lambda s: (0, 0)),
            pl.BlockSpec(memory_space=pltpu.SMEM),
            pl.BlockSpec(memory_space=pltpu.SMEM),
        ],
        out_specs=pl.BlockSpec((G * B4, J * H), lambda s: (s, 0)),
        out_shape=jax.ShapeDtypeStruct((F * B4, J * H), jnp.float32),
        scratch_shapes=[
            pltpu.VMEM((B, NW), jnp.float32),
            pltpu.VMEM((8 * J * G, K), jnp.float32),
            pltpu.VMEM((K, NW), jnp.float32),
            pltpu.VMEM((NW, NW), jnp.float32),
            pltpu.VMEM((NW, NW), jnp.float32),
        ],
        compiler_params=pltpu.CompilerParams(
            dimension_semantics=("arbitrary",),
        ),
        name="feature_loo_mlp_perm",
    )(Ap, Ashp, W1, b1c, W2, b2c, W3, b3c, bfrow, rp, basearr, thrarr)

    return out.reshape(-1)


# speedup vs baseline: 8.9335x; 1.0018x over previous
import jax
import jax.numpy as jnp
from jax import lax
from jax.experimental import pallas as pl
from jax.experimental.pallas import tpu as pltpu


def _pick_group(F, H):
    for d in range(1, F + 1):
        if F % d == 0 and d * H >= 256:
            return d
    return F


def _body(a_ref, ash_ref, w1_ref, b1_ref, w2_ref, b2_ref, w3_ref, b3_ref,
          bfrow_ref, rp_ref, base_ref, thr_ref, o_ref,
          h1s_ref, xfix_ref, w1c_ref, w2d_ref, w3d_ref):
    s = pl.program_id(0)
    B, K = a_ref.shape
    G = w1_ref.shape[0]
    H = w1_ref.shape[2]
    NW = G * H
    J = min(128 // H, B // 8)
    B4 = B // J

    w1c_ref[...] = jnp.concatenate([w1_ref[g] for g in range(G)], axis=1)

    @pl.when(s == 0)
    def _zero_diag():
        w2d_ref[...] = jnp.zeros_like(w2d_ref)
        w3d_ref[...] = jnp.zeros_like(w3d_ref)

    for g in range(G):
        w2d_ref[g * H:(g + 1) * H, g * H:(g + 1) * H] = w2_ref[g]
        w3d_ref[g * H:(g + 1) * H, g * H:(g + 1) * H] = w3_ref[g]

    w1c = w1c_ref[...].astype(jnp.bfloat16)
    p = jnp.dot(a_ref[...].astype(jnp.bfloat16), w1c,
                preferred_element_type=jnp.float32)
    q = jnp.dot(ash_ref[...].astype(jnp.bfloat16), w1c,
                preferred_element_type=jnp.float32)
    bf = bfrow_ref[0]
    rp = rp_ref[...]
    h1s_ref[...] = jnp.where(rp < bf, p, q)

    for g in range(G):
        base = pl.multiple_of(base_ref[s, g], 8 * J)
        thr = thr_ref[s, g]
        r0 = pl.multiple_of(base // J, 8)
        for j in range(J):
            apj = a_ref[pl.ds(r0 + B4 * j, 8), :]
            ashpj = ash_ref[pl.ds(r0 + B4 * j, 8), :]
            borig = base + J * lax.broadcasted_iota(jnp.int32, (8, K), 0) + j
            n8 = borig * K + lax.broadcasted_iota(jnp.int32, (8, K), 1)
            xfix_ref[8 * (J * g + j):8 * (J * g + j + 1), :] = (
                jnp.where(n8 < thr, apj, ashpj))
    fix = jnp.dot(xfix_ref[...].astype(jnp.bfloat16), w1c,
                  preferred_element_type=jnp.float32)
    lane = lax.broadcasted_iota(jnp.int32, (8, NW), 1)
    for g in range(G):
        base = pl.multiple_of(base_ref[s, g], 8 * J)
        r0 = pl.multiple_of(base // J, 8)
        m = (lane >= g * H) & (lane < (g + 1) * H)
        for j in range(J):
            win = h1s_ref[pl.ds(r0 + B4 * j, 8), :]
            h1s_ref[pl.ds(r0 + B4 * j, 8), :] = jnp.where(
                m, fix[8 * (J * g + j):8 * (J * g + j + 1), :], win)

    h = jnp.maximum(h1s_ref[...] + b1_ref[0], 0.0)
    h = jnp.dot(h.astype(jnp.bfloat16), w2d_ref[...].astype(jnp.bfloat16),
                preferred_element_type=jnp.float32) + b2_ref[0]
    h = jnp.maximum(h, 0.0)
    h = jnp.dot(h.astype(jnp.bfloat16), w3d_ref[...].astype(jnp.bfloat16),
                preferred_element_type=jnp.float32) + b3_ref[0]
    h = jnp.maximum(h, 0.0)

    for g in range(G):
        t = jnp.concatenate(
            [h[B4 * j:B4 * (j + 1), g * H:(g + 1) * H] for j in range(J)],
            axis=1)
        o_ref[g * B4:(g + 1) * B4, :] = t


def kernel(x, W1, b1, W2, b2, W3, b3):
    F, B = x.shape
    K = F - 1
    H = b1.shape[-1]
    G = _pick_group(F, H)
    S = F // G
    NW = G * H
    J = min(128 // H, B // 8)
    B4 = B // J

    xf = x.reshape(-1)
    A = xf[:B * K].reshape(B, K)
    Ash = xf[B:B + B * K].reshape(B, K)
    Ap = A.reshape(B4, J, K).transpose(1, 0, 2).reshape(B, K)
    Ashp = Ash.reshape(B4, J, K).transpose(1, 0, 2).reshape(B, K)
    permvec = (J * (jnp.arange(B, dtype=jnp.int32) % B4)
               + jnp.arange(B, dtype=jnp.int32) // B4)
    rp = jnp.broadcast_to(permvec[:, None], (B, NW))

    b1c = b1.reshape(S, 1, NW)
    b2c = b2.reshape(S, 1, NW)
    b3c = b3.reshape(S, 1, NW)

    f_all = jnp.arange(F, dtype=jnp.int32)
    t_all = f_all * B
    bf_all = t_all // K
    bfrow = jnp.repeat(bf_all.reshape(S, G), H, axis=1).reshape(S, 1, NW)
    W8 = 8 * J
    basearr = jnp.minimum((bf_all // W8) * W8, B - W8).reshape(S, G)
    thrarr = t_all.reshape(S, G)

    out = pl.pallas_call(
        _body,
        grid=(S,),
        in_specs=[
            pl.BlockSpec((B, K), lambda s: (0, 0)),
            pl.BlockSpec((B, K), lambda s: (0, 0)),
            pl.BlockSpec((G, K, H), lambda s: (s, 0, 0)),
            pl.BlockSpec((1, 1, NW), lambda s: (s, 0, 0)),
            pl.BlockSpec((G, H, H), lambda s: (s, 0, 0)),
            pl.BlockSpec((1, 1, NW), lambda s: (s, 0, 0)),
            pl.BlockSpec((G, H, H), lambda s: (s, 0, 0)),
            pl.BlockSpec((1, 1, NW), lambda s: (s, 0, 0)),
            pl.BlockSpec((1, 1, NW), lambda s: (s, 0, 0)),
            pl.BlockSpec((B, NW), lambda s: (0, 0)),
            pl.BlockSpec(memory_space=pltpu.SMEM),
            pl.BlockSpec(memory_space=pltpu.SMEM),
        ],
        out_specs=pl.BlockSpec((G * B4, J * H), lambda s: (s, 0)),
        out_shape=jax.ShapeDtypeStruct((F * B4, J * H), jnp.float32),
        scratch_shapes=[
            pltpu.VMEM((B, NW), jnp.float32),
            pltpu.VMEM((8 * J * G, K), jnp.float32),
            pltpu.VMEM((K, NW), jnp.float32),
            pltpu.VMEM((NW, NW), jnp.float32),
            pltpu.VMEM((NW, NW), jnp.float32),
        ],
        compiler_params=pltpu.CompilerParams(
            dimension_semantics=("arbitrary",),
        ),
        name="feature_loo_mlp_perm",
    )(Ap, Ashp, W1, b1c, W2, b2c, W3, b3c, bfrow, rp, basearr, thrarr)

    return out.reshape(-1)
